# Initial kernel scaffold; baseline (speedup 1.0000x reference)
import jax
import jax.numpy as jnp
from jax.experimental import pallas as pl


def kernel(x, emb, W1, b1, W2, b2, Wout, bout):
    raise NotImplementedError("write your pallas kernel here")


# speedup vs baseline: 41.9196x; 41.9196x over previous
import functools

import jax
import jax.numpy as jnp
from jax import lax
from jax.experimental import pallas as pl
from jax.experimental.pallas import tpu as pltpu
from jax.experimental.pallas import tpu_sc as plsc

B = 16384
F = 100
D = 16
H1, H2, H3 = 16, 64, 16
NCLS = 2

NC = 2
NS = 16
NW = NC * NS
BPW = B // NW
S = 32
NCHUNK = BPW // S
ROWS = S * F
GSUB = 128
NSUB = ROWS // GSUB

_mesh = plsc.VectorSubcoreMesh(
    core_axis_name="c", subcore_axis_name="s", num_cores=NC, num_subcores=NS
)


IPW = BPW * F // GSUB

_SC_SCRATCH = [
    pltpu.VMEM((IPW, GSUB), jnp.int32),
    pltpu.VMEM((ROWS, D), jnp.float32),
    pltpu.VMEM((S, D), jnp.float32),
    pltpu.SemaphoreType.DMA,
]


def _pooled_sc_body(xf_hbm, emb_hbm, out_hbm, idx_v, rows_v, pooled_v, sem):
    wid = lax.axis_index("s") * NC + lax.axis_index("c")
    row_base = wid * BPW

    pltpu.sync_copy(xf_hbm.at[pl.ds(wid * IPW, IPW)], idx_v)

    def chunk_body(c, _):
        copies = []
        for k in range(NSUB):
            copies.append(
                pltpu.async_copy(
                    emb_hbm.at[idx_v.at[c * NSUB + k]],
                    rows_v.at[pl.ds(k * GSUB, GSUB), :],
                    sem,
                )
            )
        for cp in copies:
            cp.wait()

        def sample_body(s, _):
            r0 = s * F
            a0 = rows_v[r0 + 0, :]
            a1 = rows_v[r0 + 1, :]
            a2 = rows_v[r0 + 2, :]
            a3 = rows_v[r0 + 3, :]
            for f in range(4, F, 4):
                a0 = a0 + rows_v[r0 + f + 0, :]
                a1 = a1 + rows_v[r0 + f + 1, :]
                a2 = a2 + rows_v[r0 + f + 2, :]
                a3 = a3 + rows_v[r0 + f + 3, :]
            pooled_v[s, :] = (a0 + a1) + (a2 + a3)
            return _

        lax.fori_loop(0, S, sample_body, None)
        pltpu.sync_copy(pooled_v, out_hbm.at[pl.ds(row_base + c * S, S)])
        return _

    lax.fori_loop(0, NCHUNK, chunk_body, None)


_pooled_sc = pl.kernel(
    _pooled_sc_body,
    out_type=jax.ShapeDtypeStruct((B, D), jnp.float32),
    mesh=_mesh,
    scratch_types=_SC_SCRATCH,
    compiler_params=pltpu.CompilerParams(use_tc_tiling_on_sc=False),
)


def _head_body(p_ref, w1_ref, b1_ref, w2_ref, b2_ref, wo_ref, bo_ref, o_ref):
    h = p_ref[...]
    h = jnp.dot(h, w1_ref[...], preferred_element_type=jnp.float32) + b1_ref[...]
    h = jnp.dot(h, w2_ref[...], preferred_element_type=jnp.float32) + b2_ref[...]
    h = jnp.dot(h, wo_ref[...], preferred_element_type=jnp.float32) + bo_ref[...]
    o_ref[...] = jax.nn.sigmoid(h)


_HBLK = 4096


def _head(pooled, W1, b1, W2, b2, Wout, bout):
    grid = (B // _HBLK,)
    return pl.pallas_call(
        _head_body,
        grid=grid,
        in_specs=[
            pl.BlockSpec((_HBLK, D), lambda i: (i, 0)),
            pl.BlockSpec((H1, H2), lambda i: (0, 0)),
            pl.BlockSpec((1, H2), lambda i: (0, 0)),
            pl.BlockSpec((H2, H3), lambda i: (0, 0)),
            pl.BlockSpec((1, H3), lambda i: (0, 0)),
            pl.BlockSpec((H3, NCLS), lambda i: (0, 0)),
            pl.BlockSpec((1, NCLS), lambda i: (0, 0)),
        ],
        out_specs=pl.BlockSpec((_HBLK, NCLS), lambda i: (i, 0)),
        out_shape=jax.ShapeDtypeStruct((B, NCLS), jnp.float32),
    )(
        pooled,
        W1,
        b1.reshape(1, H2),
        W2,
        b2.reshape(1, H3),
        Wout,
        bout.reshape(1, NCLS),
    )


def kernel(x, emb, W1, b1, W2, b2, Wout, bout):
    xf = x.reshape(B * F // GSUB, GSUB)
    pooled = _pooled_sc(xf, emb)
    return _head(pooled, W1, b1, W2, b2, Wout, bout)


# speedup vs baseline: 44.8689x; 1.0704x over previous
import jax
import jax.numpy as jnp
from jax import lax
from jax.experimental import pallas as pl
from jax.experimental.pallas import tpu as pltpu
from jax.experimental.pallas import tpu_sc as plsc

B = 16384
F = 100
D = 16
H1, H2, H3 = 16, 64, 16
NCLS = 2

NC = 2
NS = 16
NW = NC * NS
BPW = B // NW
RING = 16
NOUTER = BPW // RING

_mesh = plsc.VectorSubcoreMesh(
    core_axis_name="c", subcore_axis_name="s", num_cores=NC, num_subcores=NS
)

_SC_SCRATCH = (
    [pltpu.VMEM((BPW, F), jnp.int32)]
    + [pltpu.VMEM((F, D), jnp.float32) for _ in range(RING)]
    + [pltpu.VMEM((BPW, D), jnp.float32)]
    + [pltpu.SemaphoreType.DMA for _ in range(RING)]
)


def _pooled_sc_body(x_hbm, emb_hbm, out_hbm, idx_v, *rest):
    bufs = rest[:RING]
    pooled_v = rest[RING]
    sems = rest[RING + 1 : 2 * RING + 1]

    wid = lax.axis_index("s") * NC + lax.axis_index("c")
    row_base = wid * BPW

    pltpu.sync_copy(x_hbm.at[pl.ds(row_base, BPW)], idx_v)

    for r in range(RING):
        pltpu.async_copy(emb_hbm.at[idx_v.at[r]], bufs[r], sems[r])

    def outer_body(g, _):
        for r in range(RING):
            s = g * RING + r
            pltpu.make_async_copy(emb_hbm.at[idx_v.at[s]], bufs[r], sems[r]).wait()

            rows_v = bufs[r]
            a0 = rows_v[0, :]
            a1 = rows_v[1, :]
            a2 = rows_v[2, :]
            a3 = rows_v[3, :]
            for f in range(4, F, 4):
                a0 = a0 + rows_v[f + 0, :]
                a1 = a1 + rows_v[f + 1, :]
                a2 = a2 + rows_v[f + 2, :]
                a3 = a3 + rows_v[f + 3, :]
            pooled_v[s, :] = (a0 + a1) + (a2 + a3)

            @pl.when(g + 1 < NOUTER)
            def _refill():
                pltpu.async_copy(emb_hbm.at[idx_v.at[s + RING]], bufs[r], sems[r])
        return _

    lax.fori_loop(0, NOUTER, outer_body, None)
    pltpu.sync_copy(pooled_v, out_hbm.at[pl.ds(row_base, BPW)])


_pooled_sc = pl.kernel(
    _pooled_sc_body,
    out_type=jax.ShapeDtypeStruct((B, D), jnp.float32),
    mesh=_mesh,
    scratch_types=_SC_SCRATCH,
    compiler_params=pltpu.CompilerParams(use_tc_tiling_on_sc=False),
)


def _head_body(p_ref, w1_ref, b1_ref, w2_ref, b2_ref, wo_ref, bo_ref, o_ref):
    h = p_ref[...]
    h = jnp.dot(h, w1_ref[...], preferred_element_type=jnp.float32) + b1_ref[...]
    h = jnp.dot(h, w2_ref[...], preferred_element_type=jnp.float32) + b2_ref[...]
    h = jnp.dot(h, wo_ref[...], preferred_element_type=jnp.float32) + bo_ref[...]
    o_ref[...] = jax.nn.sigmoid(h)


_HBLK = 4096


def _head(pooled, W1, b1, W2, b2, Wout, bout):
    grid = (B // _HBLK,)
    return pl.pallas_call(
        _head_body,
        grid=grid,
        in_specs=[
            pl.BlockSpec((_HBLK, D), lambda i: (i, 0)),
            pl.BlockSpec((H1, H2), lambda i: (0, 0)),
            pl.BlockSpec((1, H2), lambda i: (0, 0)),
            pl.BlockSpec((H2, H3), lambda i: (0, 0)),
            pl.BlockSpec((1, H3), lambda i: (0, 0)),
            pl.BlockSpec((H3, NCLS), lambda i: (0, 0)),
            pl.BlockSpec((1, NCLS), lambda i: (0, 0)),
        ],
        out_specs=pl.BlockSpec((_HBLK, NCLS), lambda i: (i, 0)),
        out_shape=jax.ShapeDtypeStruct((B, NCLS), jnp.float32),
    )(
        pooled,
        W1,
        b1.reshape(1, H2),
        W2,
        b2.reshape(1, H3),
        Wout,
        bout.reshape(1, NCLS),
    )


def kernel(x, emb, W1, b1, W2, b2, Wout, bout):
    pooled = _pooled_sc(x, emb)
    return _head(pooled, W1, b1, W2, b2, Wout, bout)
